# Initial kernel scaffold; baseline (speedup 1.0000x reference)
import jax
import jax.numpy as jnp
from jax.experimental import pallas as pl


def kernel(context_features, context_labels):
    raise NotImplementedError("write your pallas kernel here")


# speedup vs baseline: 2.1690x; 2.1690x over previous
import functools

import jax
import jax.numpy as jnp
from jax import lax
from jax.experimental import pallas as pl
from jax.experimental.pallas import tpu as pltpu
from jax.experimental.pallas import tpu_sc as plsc

N = 160000
D = 256
C = 64
NC = 2
NS = 16
NW = NC * NS
ROWS_PER_W = N // NW
CHUNK = 128
NFULL = ROWS_PER_W // CHUNK
TAIL = ROWS_PER_W - NFULL * CHUNK
CNT_LANES = 16
LANE = 16


def _sc_partial_sums(features, labels):
    mesh = plsc.VectorSubcoreMesh(
        core_axis_name="c", subcore_axis_name="s", num_cores=NC, num_subcores=NS
    )

    @functools.partial(
        pl.kernel,
        out_type=(
            jax.ShapeDtypeStruct((NW, C, D), jnp.float32),
            jax.ShapeDtypeStruct((NW, C, CNT_LANES), jnp.float32),
        ),
        mesh=mesh,
        scratch_types=[
            pltpu.VMEM((CHUNK, D), jnp.float32),
            pltpu.VMEM((ROWS_PER_W + LANE,), jnp.int32),
            pltpu.VMEM((C, D), jnp.float32),
            pltpu.VMEM((C, CNT_LANES), jnp.float32),
        ],
    )
    def k(feat_hbm, lab_hbm, out_sum, out_cnt, buf, labv, acc, cnt):
        cid = lax.axis_index("c")
        sid = lax.axis_index("s")
        wid = cid * NS + sid
        base = wid * ROWS_PER_W

        zero16 = jnp.zeros((LANE,), jnp.float32)
        one16 = jnp.ones((LANE,), jnp.float32)

        def zero_acc(i, _):
            r = i // (D // LANE)
            j = i - r * (D // LANE)
            acc[r, pl.ds(j * LANE, LANE)] = zero16
            return 0

        lax.fori_loop(0, C * (D // LANE), zero_acc, 0)

        def zero_cnt(i, _):
            cnt[i, :] = zero16
            return 0

        lax.fori_loop(0, C, zero_cnt, 0)

        pltpu.sync_copy(lab_hbm.at[pl.ds(base, ROWS_PER_W)],
                        labv.at[pl.ds(0, ROWS_PER_W)])

        def row_body(row, _):
            lab = labv[pl.ds(row, LANE)][0]
            r = row - (row // CHUNK) * CHUNK
            for j in range(D // LANE):
                plsc.addupdate(acc.at[lab, pl.ds(j * LANE, LANE)],
                               buf[r, pl.ds(j * LANE, LANE)])
            plsc.addupdate(cnt.at[lab, :], one16)
            return 0

        def chunk_body(i, _):
            r0 = base + i * CHUNK
            pltpu.sync_copy(feat_hbm.at[pl.ds(r0, CHUNK)], buf)
            lax.fori_loop(i * CHUNK, i * CHUNK + CHUNK, row_body, 0)
            return 0

        lax.fori_loop(0, NFULL, chunk_body, 0)

        t0 = base + NFULL * CHUNK
        pltpu.sync_copy(feat_hbm.at[pl.ds(t0, TAIL)], buf.at[pl.ds(0, TAIL)])
        lax.fori_loop(NFULL * CHUNK, NFULL * CHUNK + TAIL, row_body, 0)

        pltpu.sync_copy(acc, out_sum.at[wid])
        pltpu.sync_copy(cnt, out_cnt.at[wid])

    return k(features, labels)


def _finalize(psum, pcnt):
    def body(ps_ref, pc_ref, out_ref):
        s = jnp.sum(ps_ref[...], axis=0)
        c = jnp.sum(pc_ref[...], axis=0)[:, 0:1]
        out_ref[...] = s / jnp.maximum(c, 1.0)

    return pl.pallas_call(
        body,
        out_shape=jax.ShapeDtypeStruct((C, D), jnp.float32),
    )(psum, pcnt)


def kernel(context_features, context_labels):
    labels = context_labels.astype(jnp.int32)
    psum, pcnt = _sc_partial_sums(context_features, labels)
    return _finalize(psum, pcnt)


# speedup vs baseline: 5.8506x; 2.6974x over previous
import functools

import jax
import jax.numpy as jnp
from jax import lax
from jax.experimental import pallas as pl
from jax.experimental.pallas import tpu as pltpu
from jax.experimental.pallas import tpu_sc as plsc

N = 160000
D = 256
C = 64
NC = 2
NS = 16
NW = NC * NS
CHUNK = 128
BLOCK = 64
NFULL = 39
ROWS_PER_W = NFULL * CHUNK
EXTRA = (N - NW * ROWS_PER_W) // CHUNK
EXTRA_BASE = NW * ROWS_PER_W
CNT_LANES = 16
LANE = 16
NV = D // LANE
LAB_PAD = ROWS_PER_W + CHUNK + LANE


def _sc_partial_sums(features, labels):
    mesh = plsc.VectorSubcoreMesh(
        core_axis_name="c", subcore_axis_name="s", num_cores=NC, num_subcores=NS
    )

    @functools.partial(
        pl.kernel,
        out_type=(
            jax.ShapeDtypeStruct((NW, C, D), jnp.float32),
            jax.ShapeDtypeStruct((NW, C, CNT_LANES), jnp.float32),
        ),
        mesh=mesh,
        scratch_types=[
            pltpu.VMEM((2 * CHUNK, D), jnp.float32),
            pltpu.VMEM((LAB_PAD,), jnp.int32),
            pltpu.VMEM((C, D), jnp.float32),
            pltpu.VMEM((C, CNT_LANES), jnp.float32),
            pltpu.SemaphoreType.DMA,
            pltpu.SemaphoreType.DMA,
        ],
    )
    def k(feat_hbm, lab_hbm, out_sum, out_cnt, buf, labv, acc, cnt, sem0, sem1):
        cid = lax.axis_index("c")
        sid = lax.axis_index("s")
        wid = cid * NS + sid
        base = wid * ROWS_PER_W

        zero16 = jnp.zeros((LANE,), jnp.float32)
        one16 = jnp.ones((LANE,), jnp.float32)
        blkN = jnp.full((LANE,), float(BLOCK), jnp.float32)

        def zero_acc(i, _):
            r = i // NV
            j = i - r * NV
            acc[r, pl.ds(j * LANE, LANE)] = zero16
            return 0

        lax.fori_loop(0, C * NV, zero_acc, 0)

        def zero_cnt(i, _):
            cnt[i, :] = zero16
            return 0

        lax.fori_loop(0, C, zero_cnt, 0)

        pltpu.sync_copy(lab_hbm.at[pl.ds(base, ROWS_PER_W)],
                        labv.at[pl.ds(0, ROWS_PER_W)])

        @pl.when(wid < EXTRA)
        def _():
            pltpu.sync_copy(lab_hbm.at[pl.ds(EXTRA_BASE + wid * CHUNK, CHUNK)],
                            labv.at[pl.ds(ROWS_PER_W, CHUNK)])

        def process(boff, lab_base):
            def sub_body(sb, _):
                bb = sb * BLOCK
                lb = lab_base + bb
                rb = boff + bb
                w0 = labv[pl.ds(lb, LANE)][0]
                wL = labv[pl.ds(lb + BLOCK - LANE, LANE)][LANE - 1]

                @pl.when(w0 == wL)
                def _():
                    for j in range(NV):
                        stack = []
                        for r in range(BLOCK):
                            v = buf[rb + r, pl.ds(j * LANE, LANE)]
                            lvl = 0
                            while stack and stack[-1][0] == lvl:
                                v = stack.pop()[1] + v
                                lvl += 1
                            stack.append((lvl, v))
                        tot = stack.pop()[1]
                        while stack:
                            tot = tot + stack.pop()[1]
                        plsc.addupdate(acc.at[w0, pl.ds(j * LANE, LANE)], tot)
                    plsc.addupdate(cnt.at[w0, :], blkN)

                @pl.when(w0 != wL)
                def _():
                    def row_body(r, _):
                        lab = labv[pl.ds(lb + r, LANE)][0]
                        for j in range(NV):
                            plsc.addupdate(
                                acc.at[lab, pl.ds(j * LANE, LANE)],
                                buf[rb + r, pl.ds(j * LANE, LANE)])
                        plsc.addupdate(cnt.at[lab, :], one16)
                        return 0

                    lax.fori_loop(0, BLOCK, row_body, 0)

                return 0

            lax.fori_loop(0, CHUNK // BLOCK, sub_body, 0)

        def feat_slice(r0):
            return feat_hbm.at[pl.ds(r0, CHUNK)]

        def buf_at(boff):
            return buf.at[pl.ds(boff, CHUNK)]

        pltpu.async_copy(feat_slice(base), buf_at(0), sem0)

        def chunk_pair(p, _):
            c0 = 2 * p * CHUNK
            pltpu.make_async_copy(feat_slice(base + c0), buf_at(0), sem0).wait()
            pltpu.async_copy(feat_slice(base + c0 + CHUNK), buf_at(CHUNK), sem1)
            process(0, c0)
            pltpu.make_async_copy(
                feat_slice(base + c0 + CHUNK), buf_at(CHUNK), sem1).wait()
            pltpu.async_copy(feat_slice(base + c0 + 2 * CHUNK), buf_at(0), sem0)
            process(CHUNK, c0 + CHUNK)
            return 0

        lax.fori_loop(0, NFULL // 2, chunk_pair, 0)

        last = (NFULL - 1) * CHUNK
        pltpu.make_async_copy(feat_slice(base + last), buf_at(0), sem0).wait()

        @pl.when(wid < EXTRA)
        def _():
            pltpu.async_copy(
                feat_slice(EXTRA_BASE + wid * CHUNK), buf_at(CHUNK), sem1)

        def tail_body(t, _):
            @pl.when((t == 0) | (wid < EXTRA))
            def _():
                @pl.when(t == 1)
                def _():
                    pltpu.make_async_copy(
                        feat_slice(EXTRA_BASE + wid * CHUNK), buf_at(CHUNK),
                        sem1).wait()

                process(t * CHUNK, jnp.where(t == 0, last, ROWS_PER_W))

            return 0

        lax.fori_loop(0, 2, tail_body, 0)

        pltpu.sync_copy(acc, out_sum.at[wid])
        pltpu.sync_copy(cnt, out_cnt.at[wid])

    return k(features, labels)


def _finalize(psum, pcnt):
    def body(ps_ref, pc_ref, out_ref):
        s = jnp.sum(ps_ref[...], axis=0)
        c = jnp.sum(pc_ref[...], axis=0)[:, 0:1]
        out_ref[...] = s / jnp.maximum(c, 1.0)

    return pl.pallas_call(
        body,
        out_shape=jax.ShapeDtypeStruct((C, D), jnp.float32),
    )(psum, pcnt)


def kernel(context_features, context_labels):
    labels = context_labels.astype(jnp.int32)
    psum, pcnt = _sc_partial_sums(context_features, labels)
    return _finalize(psum, pcnt)


# speedup vs baseline: 6.6266x; 1.1326x over previous
import functools

import jax
import jax.numpy as jnp
from jax import lax
from jax.experimental import pallas as pl
from jax.experimental.pallas import tpu as pltpu
from jax.experimental.pallas import tpu_sc as plsc

N = 160000
D = 256
C = 64
NC = 2
NS = 16
NW = NC * NS
CHUNK = 128
BLOCK = 32
NFULL = 20
ROWS_PER_W = NFULL * CHUNK
SC_N = NW * ROWS_PER_W
TC_N = N - SC_N
RB = 1280
TC_G = TC_N // RB
CNT_LANES = 16
LANE = 16
NV = D // LANE
LAB_PAD = ROWS_PER_W + LANE


def _sc_partial_sums(features, labels):
    mesh = plsc.VectorSubcoreMesh(
        core_axis_name="c", subcore_axis_name="s", num_cores=NC, num_subcores=NS
    )

    @functools.partial(
        pl.kernel,
        out_type=(
            jax.ShapeDtypeStruct((NW, C, D), jnp.float32),
            jax.ShapeDtypeStruct((NW, C, CNT_LANES), jnp.float32),
        ),
        mesh=mesh,
        scratch_types=[
            pltpu.VMEM((CHUNK, D), jnp.float32),
            pltpu.VMEM((CHUNK, D), jnp.float32),
            pltpu.VMEM((LAB_PAD,), jnp.int32),
            pltpu.VMEM((C, D), jnp.float32),
            pltpu.VMEM((C, CNT_LANES), jnp.float32),
            pltpu.SemaphoreType.DMA,
            pltpu.SemaphoreType.DMA,
        ],
    )
    def k(feat_hbm, lab_hbm, out_sum, out_cnt, buf0, buf1, labv, acc, cnt,
          sem0, sem1):
        cid = lax.axis_index("c")
        sid = lax.axis_index("s")
        wid = cid * NS + sid
        base = wid * ROWS_PER_W

        zero16 = jnp.zeros((LANE,), jnp.float32)
        one16 = jnp.ones((LANE,), jnp.float32)
        blkN = jnp.full((LANE,), float(BLOCK), jnp.float32)

        def zero_acc(i, _):
            r = i // NV
            j = i - r * NV
            acc[r, pl.ds(j * LANE, LANE)] = zero16
            return 0

        lax.fori_loop(0, C * NV, zero_acc, 0)

        def zero_cnt(i, _):
            cnt[i, :] = zero16
            return 0

        lax.fori_loop(0, C, zero_cnt, 0)

        pltpu.sync_copy(lab_hbm.at[pl.ds(base, ROWS_PER_W)],
                        labv.at[pl.ds(0, ROWS_PER_W)])

        def process(buf, lab_base):
            def sub_body(sb, _):
                bb = sb * BLOCK
                lb = lab_base + bb
                rb = bb
                w0 = labv[pl.ds(lb, LANE)][0]
                wL = labv[pl.ds(lb + BLOCK - LANE, LANE)][LANE - 1]

                @pl.when(w0 == wL)
                def _():
                    for j in range(NV):
                        stack = []
                        for r in range(BLOCK):
                            v = buf[rb + r, pl.ds(j * LANE, LANE)]
                            lvl = 0
                            while stack and stack[-1][0] == lvl:
                                v = stack.pop()[1] + v
                                lvl += 1
                            stack.append((lvl, v))
                        tot = stack.pop()[1]
                        while stack:
                            tot = tot + stack.pop()[1]
                        plsc.addupdate(acc.at[w0, pl.ds(j * LANE, LANE)], tot)
                    plsc.addupdate(cnt.at[w0, :], blkN)

                @pl.when(w0 != wL)
                def _():
                    def row_body(r, _):
                        lab = labv[pl.ds(lb + r, LANE)][0]
                        for j in range(NV):
                            plsc.addupdate(
                                acc.at[lab, pl.ds(j * LANE, LANE)],
                                buf[rb + r, pl.ds(j * LANE, LANE)])
                        plsc.addupdate(cnt.at[lab, :], one16)
                        return 0

                    lax.fori_loop(0, BLOCK, row_body, 0)

                return 0

            lax.fori_loop(0, CHUNK // BLOCK, sub_body, 0)

        def feat_slice(r0):
            return feat_hbm.at[pl.ds(r0, CHUNK)]

        pltpu.async_copy(feat_slice(base), buf0, sem0)

        def chunk_pair(p, _):
            c0 = 2 * p * CHUNK
            nxt = jnp.where(c0 + 2 * CHUNK < ROWS_PER_W, c0 + 2 * CHUNK, 0)
            pltpu.make_async_copy(feat_slice(base + c0), buf0, sem0).wait()
            pltpu.async_copy(feat_slice(base + c0 + CHUNK), buf1, sem1)
            process(buf0, c0)
            pltpu.make_async_copy(
                feat_slice(base + c0 + CHUNK), buf1, sem1).wait()
            pltpu.async_copy(feat_slice(base + nxt), buf0, sem0)
            process(buf1, c0 + CHUNK)
            return 0

        lax.fori_loop(0, NFULL // 2, chunk_pair, 0)

        pltpu.make_async_copy(feat_slice(base), buf0, sem0).wait()

        pltpu.sync_copy(acc, out_sum.at[wid])
        pltpu.sync_copy(cnt, out_cnt.at[wid])

    return k(features, labels)


def _tc_partial_sums(features_tc, labels_tc3):
    def body(lab_ref, feat_ref, sum_ref, cnt_ref):
        lab = lab_ref[0, 0, :]
        oh = (lax.broadcasted_iota(jnp.int32, (C, RB), 0)
              == lab[None, :]).astype(jnp.float32)
        ps = lax.dot_general(oh, feat_ref[...], (((1,), (0,)), ((), ())),
                             preferred_element_type=jnp.float32)
        pc = jnp.sum(oh, axis=1, keepdims=True)

        @pl.when(pl.program_id(0) == 0)
        def _():
            sum_ref[...] = jnp.zeros_like(sum_ref)
            cnt_ref[...] = jnp.zeros_like(cnt_ref)

        sum_ref[...] += ps
        cnt_ref[...] += jnp.broadcast_to(pc, cnt_ref.shape)

    return pl.pallas_call(
        body,
        grid=(TC_G,),
        in_specs=[
            pl.BlockSpec((1, 1, RB), lambda g: (g, 0, 0)),
            pl.BlockSpec((RB, D), lambda g: (g, 0)),
        ],
        out_specs=[
            pl.BlockSpec((C, D), lambda g: (0, 0)),
            pl.BlockSpec((C, 128), lambda g: (0, 0)),
        ],
        out_shape=[
            jax.ShapeDtypeStruct((C, D), jnp.float32),
            jax.ShapeDtypeStruct((C, 128), jnp.float32),
        ],
    )(labels_tc3, features_tc)


def _finalize(psum, pcnt, tsum, tcnt):
    def body(ps_ref, pc_ref, ts_ref, tc_ref, out_ref):
        s = jnp.sum(ps_ref[...], axis=0) + ts_ref[...]
        c = jnp.sum(pc_ref[...], axis=0)[:, 0:1] + tc_ref[...][:, 0:1]
        out_ref[...] = s / jnp.maximum(c, 1.0)

    return pl.pallas_call(
        body,
        out_shape=jax.ShapeDtypeStruct((C, D), jnp.float32),
    )(psum, pcnt, tsum, tcnt)


def kernel(context_features, context_labels):
    labels = context_labels.astype(jnp.int32)
    psum, pcnt = _sc_partial_sums(context_features, labels)
    tsum, tcnt = _tc_partial_sums(
        context_features[SC_N:], labels[SC_N:].reshape(TC_G, 1, RB))
    return _finalize(psum, pcnt, tsum, tcnt)


# speedup vs baseline: 7.8419x; 1.1834x over previous
import functools

import jax
import jax.numpy as jnp
from jax import lax
from jax.experimental import pallas as pl
from jax.experimental.pallas import tpu as pltpu
from jax.experimental.pallas import tpu_sc as plsc

N = 160000
D = 256
C = 64
NC = 2
NS = 16
NW = NC * NS
CHUNK = 128
BLOCK = 32
NFULL = 39
ROWS_PER_W = NFULL * CHUNK
EXTRA = (N - NW * ROWS_PER_W) // CHUNK
EXTRA_BASE = NW * ROWS_PER_W
CNT_LANES = 16
LANE = 16
NV = D // LANE
LAB_PAD = ROWS_PER_W + CHUNK + LANE


def _sc_partial_sums(features, labels):
    mesh = plsc.VectorSubcoreMesh(
        core_axis_name="c", subcore_axis_name="s", num_cores=NC, num_subcores=NS
    )

    @functools.partial(
        pl.kernel,
        out_type=(
            jax.ShapeDtypeStruct((NW, C, D), jnp.float32),
            jax.ShapeDtypeStruct((NW, C, CNT_LANES), jnp.float32),
        ),
        mesh=mesh,
        scratch_types=[
            pltpu.VMEM((CHUNK, D), jnp.float32),
            pltpu.VMEM((CHUNK, D), jnp.float32),
            pltpu.VMEM((LAB_PAD,), jnp.int32),
            pltpu.VMEM((C, D), jnp.float32),
            pltpu.VMEM((C, CNT_LANES), jnp.float32),
            pltpu.SemaphoreType.DMA,
            pltpu.SemaphoreType.DMA,
        ],
    )
    def k(feat_hbm, lab_hbm, out_sum, out_cnt, buf0, buf1, labv, acc, cnt, sem0, sem1):
        cid = lax.axis_index("c")
        sid = lax.axis_index("s")
        wid = cid * NS + sid
        base = wid * ROWS_PER_W

        zero16 = jnp.zeros((LANE,), jnp.float32)
        one16 = jnp.ones((LANE,), jnp.float32)
        blkN = jnp.full((LANE,), float(BLOCK), jnp.float32)

        def zero_acc(i, _):
            r = i // NV
            j = i - r * NV
            acc[r, pl.ds(j * LANE, LANE)] = zero16
            return 0

        lax.fori_loop(0, C * NV, zero_acc, 0)

        def zero_cnt(i, _):
            cnt[i, :] = zero16
            return 0

        lax.fori_loop(0, C, zero_cnt, 0)

        pltpu.sync_copy(lab_hbm.at[pl.ds(base, ROWS_PER_W)],
                        labv.at[pl.ds(0, ROWS_PER_W)])

        @pl.when(wid < EXTRA)
        def _():
            pltpu.sync_copy(lab_hbm.at[pl.ds(EXTRA_BASE + wid * CHUNK, CHUNK)],
                            labv.at[pl.ds(ROWS_PER_W, CHUNK)])

        def process(buf, lab_base):
            def sub_body(sb, _):
                bb = sb * BLOCK
                lb = lab_base + bb
                rb = bb
                w0 = labv[pl.ds(lb, LANE)][0]
                wL = labv[pl.ds(lb + BLOCK - LANE, LANE)][LANE - 1]

                @pl.when(w0 == wL)
                def _():
                    for j in range(NV):
                        stack = []
                        for r in range(BLOCK):
                            v = buf[rb + r, pl.ds(j * LANE, LANE)]
                            lvl = 0
                            while stack and stack[-1][0] == lvl:
                                v = stack.pop()[1] + v
                                lvl += 1
                            stack.append((lvl, v))
                        tot = stack.pop()[1]
                        while stack:
                            tot = tot + stack.pop()[1]
                        plsc.addupdate(acc.at[w0, pl.ds(j * LANE, LANE)], tot)
                    plsc.addupdate(cnt.at[w0, :], blkN)

                @pl.when(w0 != wL)
                def _():
                    def row_body(r, _):
                        lab = labv[pl.ds(lb + r, LANE)][0]
                        vals = [buf[rb + r, pl.ds(j * LANE, LANE)]
                                for j in range(NV)]
                        for j in range(NV):
                            plsc.addupdate(
                                acc.at[lab, pl.ds(j * LANE, LANE)], vals[j])
                        plsc.addupdate(cnt.at[lab, :], one16)
                        return 0

                    lax.fori_loop(0, BLOCK, row_body, 0)

                return 0

            lax.fori_loop(0, CHUNK // BLOCK, sub_body, 0)

        def feat_slice(r0):
            return feat_hbm.at[pl.ds(r0, CHUNK)]

        pltpu.async_copy(feat_slice(base), buf0, sem0)

        def chunk_pair(p, _):
            c0 = 2 * p * CHUNK
            pltpu.make_async_copy(feat_slice(base + c0), buf0, sem0).wait()
            pltpu.async_copy(feat_slice(base + c0 + CHUNK), buf1, sem1)
            process(buf0, c0)
            pltpu.make_async_copy(
                feat_slice(base + c0 + CHUNK), buf1, sem1).wait()
            pltpu.async_copy(feat_slice(base + c0 + 2 * CHUNK), buf0, sem0)
            process(buf1, c0 + CHUNK)
            return 0

        lax.fori_loop(0, NFULL // 2, chunk_pair, 0)

        last = (NFULL - 1) * CHUNK
        pltpu.make_async_copy(feat_slice(base + last), buf0, sem0).wait()

        @pl.when(wid < EXTRA)
        def _():
            pltpu.async_copy(
                feat_slice(EXTRA_BASE + wid * CHUNK), buf1, sem1)

        process(buf0, last)

        @pl.when(wid < EXTRA)
        def _():
            pltpu.make_async_copy(
                feat_slice(EXTRA_BASE + wid * CHUNK), buf1, sem1).wait()
            process(buf1, ROWS_PER_W)

        pltpu.sync_copy(acc, out_sum.at[wid])
        pltpu.sync_copy(cnt, out_cnt.at[wid])

    return k(features, labels)


def _finalize(psum, pcnt):
    def body(ps_ref, pc_ref, out_ref):
        s = jnp.sum(ps_ref[...], axis=0)
        c = jnp.sum(pc_ref[...], axis=0)[:, 0:1]
        out_ref[...] = s / jnp.maximum(c, 1.0)

    return pl.pallas_call(
        body,
        out_shape=jax.ShapeDtypeStruct((C, D), jnp.float32),
    )(psum, pcnt)


def kernel(context_features, context_labels):
    labels = context_labels.astype(jnp.int32)
    psum, pcnt = _sc_partial_sums(context_features, labels)
    return _finalize(psum, pcnt)


# speedup vs baseline: 7.8973x; 1.0071x over previous
import functools

import jax
import jax.numpy as jnp
from jax import lax
from jax.experimental import pallas as pl
from jax.experimental.pallas import tpu as pltpu
from jax.experimental.pallas import tpu_sc as plsc

N = 160000
D = 256
C = 64
NC = 2
NS = 16
NW = NC * NS
CHUNK = 128
BLOCK = 32
NFULL = 39
ROWS_PER_W = NFULL * CHUNK
EXTRA_BASE = NW * ROWS_PER_W
EXTRA_N = N - EXTRA_BASE
CNT_LANES = 16
LANE = 16
NV = D // LANE
LAB_PAD = ROWS_PER_W + LANE


def _sc_partial_sums(features, labels):
    mesh = plsc.VectorSubcoreMesh(
        core_axis_name="c", subcore_axis_name="s", num_cores=NC, num_subcores=NS
    )

    @functools.partial(
        pl.kernel,
        out_type=(
            jax.ShapeDtypeStruct((NW, C, D), jnp.float32),
            jax.ShapeDtypeStruct((NW, C, CNT_LANES), jnp.float32),
        ),
        mesh=mesh,
        scratch_types=[
            pltpu.VMEM((CHUNK, D), jnp.float32),
            pltpu.VMEM((CHUNK, D), jnp.float32),
            pltpu.VMEM((LAB_PAD,), jnp.int32),
            pltpu.VMEM((C, D), jnp.float32),
            pltpu.VMEM((C, CNT_LANES), jnp.float32),
            pltpu.SemaphoreType.DMA,
            pltpu.SemaphoreType.DMA,
        ],
    )
    def k(feat_hbm, lab_hbm, out_sum, out_cnt, buf0, buf1, labv, acc, cnt, sem0, sem1):
        cid = lax.axis_index("c")
        sid = lax.axis_index("s")
        wid = cid * NS + sid
        base = wid * ROWS_PER_W

        zero16 = jnp.zeros((LANE,), jnp.float32)
        one16 = jnp.ones((LANE,), jnp.float32)
        blkN = jnp.full((LANE,), float(BLOCK), jnp.float32)

        def zero_acc(i, _):
            r = i // NV
            j = i - r * NV
            acc[r, pl.ds(j * LANE, LANE)] = zero16
            return 0

        lax.fori_loop(0, C * NV, zero_acc, 0)

        def zero_cnt(i, _):
            cnt[i, :] = zero16
            return 0

        lax.fori_loop(0, C, zero_cnt, 0)

        pltpu.sync_copy(lab_hbm.at[pl.ds(base, ROWS_PER_W)],
                        labv.at[pl.ds(0, ROWS_PER_W)])

        def process(buf, lab_base):
            def sub_body(sb, _):
                bb = sb * BLOCK
                lb = lab_base + bb
                rb = bb
                w0 = labv[pl.ds(lb, LANE)][0]
                wL = labv[pl.ds(lb + BLOCK - LANE, LANE)][LANE - 1]

                @pl.when(w0 == wL)
                def _():
                    for j in range(NV):
                        stack = []
                        for r in range(BLOCK):
                            v = buf[rb + r, pl.ds(j * LANE, LANE)]
                            lvl = 0
                            while stack and stack[-1][0] == lvl:
                                v = stack.pop()[1] + v
                                lvl += 1
                            stack.append((lvl, v))
                        tot = stack.pop()[1]
                        while stack:
                            tot = tot + stack.pop()[1]
                        plsc.addupdate(acc.at[w0, pl.ds(j * LANE, LANE)], tot)
                    plsc.addupdate(cnt.at[w0, :], blkN)

                @pl.when(w0 != wL)
                def _():
                    def row_body(r, _):
                        lab = labv[pl.ds(lb + r, LANE)][0]
                        vals = [buf[rb + r, pl.ds(j * LANE, LANE)]
                                for j in range(NV)]
                        for j in range(NV):
                            plsc.addupdate(
                                acc.at[lab, pl.ds(j * LANE, LANE)], vals[j])
                        plsc.addupdate(cnt.at[lab, :], one16)
                        return 0

                    lax.fori_loop(0, BLOCK, row_body, 0)

                return 0

            lax.fori_loop(0, CHUNK // BLOCK, sub_body, 0)

        def feat_slice(r0):
            return feat_hbm.at[pl.ds(r0, CHUNK)]

        pltpu.async_copy(feat_slice(base), buf0, sem0)

        def chunk_pair(p, _):
            c0 = 2 * p * CHUNK
            pltpu.make_async_copy(feat_slice(base + c0), buf0, sem0).wait()
            pltpu.async_copy(feat_slice(base + c0 + CHUNK), buf1, sem1)
            process(buf0, c0)
            pltpu.make_async_copy(
                feat_slice(base + c0 + CHUNK), buf1, sem1).wait()
            pltpu.async_copy(feat_slice(base + c0 + 2 * CHUNK), buf0, sem0)
            process(buf1, c0 + CHUNK)
            return 0

        lax.fori_loop(0, NFULL // 2, chunk_pair, 0)

        last = (NFULL - 1) * CHUNK
        pltpu.make_async_copy(feat_slice(base + last), buf0, sem0).wait()
        process(buf0, last)

        pltpu.sync_copy(acc, out_sum.at[wid])
        pltpu.sync_copy(cnt, out_cnt.at[wid])

    return k(features, labels)


def _finalize(psum, pcnt, extra_feats, extra_labs):
    def body(ps_ref, pc_ref, ef_ref, el_ref, out_ref):
        lab = el_ref[0, :]
        oh = (lax.broadcasted_iota(jnp.int32, (C, EXTRA_N), 0)
              == lab[None, :]).astype(jnp.float32)
        s = jnp.sum(ps_ref[...], axis=0) + lax.dot_general(
            oh, ef_ref[...], (((1,), (0,)), ((), ())),
            preferred_element_type=jnp.float32)
        c = (jnp.sum(pc_ref[...], axis=0)[:, 0:1]
             + jnp.sum(oh, axis=1, keepdims=True))
        out_ref[...] = s / jnp.maximum(c, 1.0)

    return pl.pallas_call(
        body,
        out_shape=jax.ShapeDtypeStruct((C, D), jnp.float32),
    )(psum, pcnt, extra_feats, extra_labs)


def kernel(context_features, context_labels):
    labels = context_labels.astype(jnp.int32)
    psum, pcnt = _sc_partial_sums(context_features, labels)
    return _finalize(psum, pcnt, context_features[EXTRA_BASE:],
                     labels[EXTRA_BASE:].reshape(1, EXTRA_N))
